# Initial kernel scaffold; baseline (speedup 1.0000x reference)
import jax
import jax.numpy as jnp
from jax.experimental import pallas as pl


def kernel(x, e, params, edge_index):
    raise NotImplementedError("write your pallas kernel here")


# speedup vs baseline: 2.6409x; 2.6409x over previous
import functools

import jax
import jax.numpy as jnp
from jax import lax
from jax.experimental import pallas as pl
from jax.experimental.pallas import tpu as pltpu

N_NODES = 10000
D = 128
F32 = jnp.float32


def _pre_body(h_ref, wl_ref, bl_ref, wr_ref, br_ref, xl_o, xr_o):
    h = h_ref[...]
    xl_o[...] = jnp.dot(h, wl_ref[...], preferred_element_type=F32) + bl_ref[...]
    xr_o[...] = jnp.dot(h, wr_ref[...], preferred_element_type=F32) + br_ref[...]


def _pre(h, wl, bl, wr, br):
    return pl.pallas_call(
        _pre_body,
        out_shape=[jax.ShapeDtypeStruct((N_NODES, D), F32)] * 2,
    )(h, wl, bl, wr, br)


def _combine_body(num_ref, den_ref, bias_ref, gamma_ref, beta_ref,
                  wl_ref, bl_ref, wr_ref, br_ref, xl_o, xr_o):
    num2 = num_ref[...]
    den2 = den_ref[...]
    num = num2[0] + num2[1]
    den = den2[0, :, 0:1] + den2[1, :, 0:1]
    h = num / (den + 1e-16) + bias_ref[...]
    h = jnp.tanh(h)
    mu = jnp.mean(h)
    hc = h - mu
    sd = jnp.sqrt(jnp.mean(hc * hc))
    h = hc / (sd + 1e-5)
    h = h * gamma_ref[...] + beta_ref[...]
    xl_o[...] = jnp.dot(h, wl_ref[...], preferred_element_type=F32) + bl_ref[...]
    xr_o[...] = jnp.dot(h, wr_ref[...], preferred_element_type=F32) + br_ref[...]


def _combine(num2, den2, bias, gamma, beta, wl, bl, wr, br):
    return pl.pallas_call(
        _combine_body,
        out_shape=[jax.ShapeDtypeStruct((N_NODES, D), F32)] * 2,
    )(num2, den2, bias, gamma, beta, wl, bl, wr, br)


def _final_body(num_ref, den_ref, bias_ref, out_o):
    num2 = num_ref[...]
    den2 = den_ref[...]
    num = num2[0] + num2[1]
    den = den2[0, :, 0:1] + den2[1, :, 0:1]
    out_o[...] = num / (den + 1e-16) + bias_ref[...]


def _final(num2, den2, bias):
    return pl.pallas_call(
        _final_body,
        out_shape=jax.ShapeDtypeStruct((N_NODES, D), F32),
    )(num2, den2, bias)


def _edge_stage(xl, xr, ev, src, dst, we, att):
    m = xl[src] + xr[dst] + ev[:, None] * we[None, :]
    m = jnp.where(m > 0, m, 0.2 * m)
    logits = m @ att
    ex = jnp.exp(logits)
    den = jax.ops.segment_sum(ex, dst, num_segments=N_NODES)
    num = jax.ops.segment_sum(ex[:, None] * xl[src], dst, num_segments=N_NODES)
    num2 = jnp.stack([num, jnp.zeros_like(num)])
    den16 = jnp.zeros((2, N_NODES, 16), F32).at[0, :, 0].set(den)
    return num2, den16


def kernel(x, e, params, edge_index):
    src = edge_index[0]
    dst = edge_index[1]
    ev = e[:, 0]
    gat = params['gat']
    norm = params['norm']

    def r2(v):
        return v.reshape(1, D)

    h = x
    p = gat[0]
    xl, xr = _pre(h, p['Wl'], r2(p['bl']), p['Wr'], r2(p['br']))
    for i in range(4):
        p = gat[i]
        we = p['We'][0]
        num2, den2 = _edge_stage(xl, xr, ev, src, dst, we, p['att'])
        if i < 3:
            q = gat[i + 1]
            xl, xr = _combine(num2, den2, r2(p['bias']),
                              r2(norm[i]['gamma']), r2(norm[i]['beta']),
                              q['Wl'], r2(q['bl']), q['Wr'], r2(q['br']))
        else:
            return _final(num2, den2, r2(p['bias']))


# speedup vs baseline: 3.6993x; 1.4008x over previous
import functools

import jax
import jax.numpy as jnp
from jax import lax
from jax.experimental import pallas as pl
from jax.experimental.pallas import tpu as pltpu
from jax.experimental.pallas import tpu_sc as plsc

N_NODES = 10000
D = 128
F32 = jnp.float32


def _pre_body(h_ref, wl_ref, bl_ref, wr_ref, br_ref, xl_o, xr_o):
    h = h_ref[...]
    xl_o[...] = jnp.dot(h, wl_ref[...], preferred_element_type=F32) + bl_ref[...]
    xr_o[...] = jnp.dot(h, wr_ref[...], preferred_element_type=F32) + br_ref[...]


def _pre(h, wl, bl, wr, br):
    return pl.pallas_call(
        _pre_body,
        out_shape=[jax.ShapeDtypeStruct((N_NODES, D), F32)] * 2,
    )(h, wl, bl, wr, br)


def _combine_body(num_ref, den_ref, bias_ref, gamma_ref, beta_ref,
                  wl_ref, bl_ref, wr_ref, br_ref, xl_o, xr_o):
    num = num_ref[:N_NODES, :]
    den = den_ref[:N_NODES, 0:1]
    h = num / (den + 1e-16) + bias_ref[...]
    h = jnp.tanh(h)
    mu = jnp.mean(h)
    hc = h - mu
    sd = jnp.sqrt(jnp.mean(hc * hc))
    h = hc / (sd + 1e-5)
    h = h * gamma_ref[...] + beta_ref[...]
    xl_o[...] = jnp.dot(h, wl_ref[...], preferred_element_type=F32) + bl_ref[...]
    xr_o[...] = jnp.dot(h, wr_ref[...], preferred_element_type=F32) + br_ref[...]


def _combine(num2, den2, bias, gamma, beta, wl, bl, wr, br):
    return pl.pallas_call(
        _combine_body,
        out_shape=[jax.ShapeDtypeStruct((N_NODES, D), F32)] * 2,
    )(num2, den2, bias, gamma, beta, wl, bl, wr, br)


def _final_body(num_ref, den_ref, bias_ref, out_o):
    num = num_ref[:N_NODES, :]
    den = den_ref[:N_NODES, 0:1]
    out_o[...] = num / (den + 1e-16) + bias_ref[...]


def _final(num2, den2, bias):
    return pl.pallas_call(
        _final_body,
        out_shape=jax.ShapeDtypeStruct((N_NODES, D), F32),
    )(num2, den2, bias)


NW = 16
EPW = 20000
CHUNK = 80
NCHUNK = EPW // CHUNK
N_PAD = 10240
RPT = N_PAD // 16
DROWS = N_PAD // 8
DRPT = DROWS // 16


def _sc_edge_body(xl_hbm, xr_hbm, idx3, dstc, dstdc, wv_hbm, av_hbm,
                  onum, oden,
                  idx_v, dsti_v, dstd_v, gxl, gxr, exd, w_v, a_v,
                  num_s, den_s, sem):
    sid = lax.axis_index("s")
    wid = sid

    pltpu.sync_copy(wv_hbm, w_v)
    pltpu.sync_copy(av_hbm, a_v)

    zeros16 = jnp.zeros((16,), F32)

    def zrow(r, carry):
        for j in range(8):
            gxl[r, pl.ds(16 * j, 16)] = zeros16
            exd[r, pl.ds(16 * j, 16)] = zeros16
        return carry

    lax.fori_loop(0, CHUNK, zrow, 0)

    for k in range(RPT // CHUNK):
        r0 = sid * RPT + k * CHUNK
        pltpu.sync_copy(gxl, num_s.at[pl.ds(r0, CHUNK)])
    pltpu.sync_copy(exd, den_s.at[pl.ds(sid * DRPT, DRPT)])
    plsc.subcore_barrier()

    lane = lax.broadcasted_iota(jnp.int32, (16,), 0)

    def chunk_body(c, carry):
        pltpu.sync_copy(idx3.at[wid, c], idx_v)
        pltpu.sync_copy(dstc.at[wid, c], dsti_v)
        pltpu.sync_copy(dstdc.at[wid, c], dstd_v)
        d1 = pltpu.async_copy(xl_hbm.at[idx_v.at[0]], gxl, sem)
        d2 = pltpu.async_copy(xr_hbm.at[idx_v.at[1]], gxr, sem)
        d1.wait()
        d2.wait()

        def group_body(g, carry2):
            gbase = g * 16
            evv = plsc.bitcast(idx_v[2, pl.ds(gbase, 16)], F32)
            dvv = idx_v[1, pl.ds(gbase, 16)]
            for k in range(16):
                i = gbase + k
                ev = evv[k]
                sub = dvv[k] & 7
                acc = zeros16
                asl = []
                for j in range(8):
                    av = gxl[i, pl.ds(16 * j, 16)]
                    bv = gxr[i, pl.ds(16 * j, 16)]
                    s = av + bv + ev * w_v[pl.ds(16 * j, 16)]
                    m = jnp.maximum(s, 0.2 * s)
                    acc = acc + m * a_v[pl.ds(16 * j, 16)]
                    asl.append(av)
                tot = jnp.sum(acc)
                exv = jnp.exp(jnp.full((16,), tot, F32))
                exv1 = jnp.where(lane == 0, exv, 0.0)
                for j in range(8):
                    gxl[i, pl.ds(16 * j, 16)] = exv * asl[j]
                    exd[i, pl.ds(16 * j, 16)] = jnp.where(sub == j, exv1, 0.0)
            return carry2

        lax.fori_loop(0, CHUNK // 16, group_body, 0)

        pltpu.sync_copy(gxl, num_s.at[dsti_v], add=True)
        pltpu.sync_copy(exd, den_s.at[dstd_v], add=True)
        return carry

    lax.fori_loop(0, NCHUNK, chunk_body, 0)
    plsc.subcore_barrier()

    for k in range(RPT // CHUNK):
        r0 = sid * RPT + k * CHUNK
        pltpu.sync_copy(num_s.at[pl.ds(r0, CHUNK)], gxl)
        pltpu.sync_copy(gxl, onum.at[pl.ds(r0, CHUNK)])
    pltpu.sync_copy(den_s.at[pl.ds(sid * DRPT, DRPT)], exd)
    pltpu.sync_copy(exd, oden.at[pl.ds(sid * DRPT, DRPT)])


@functools.partial(
    pl.kernel,
    out_type=[jax.ShapeDtypeStruct((N_PAD, D), F32),
              jax.ShapeDtypeStruct((DROWS, D), F32)],
    mesh=plsc.VectorSubcoreMesh(core_axis_name="c", subcore_axis_name="s",
                                num_cores=1),
    compiler_params=pltpu.CompilerParams(needs_layout_passes=False),
    scratch_types=[
        pltpu.VMEM((3, CHUNK), jnp.int32),
        pltpu.VMEM((CHUNK,), jnp.int32),
        pltpu.VMEM((CHUNK,), jnp.int32),
        pltpu.VMEM((CHUNK, D), F32),
        pltpu.VMEM((CHUNK, D), F32),
        pltpu.VMEM((CHUNK, D), F32),
        pltpu.VMEM((D,), F32),
        pltpu.VMEM((D,), F32),
        pltpu.VMEM_SHARED((N_PAD, D), F32),
        pltpu.VMEM_SHARED((DROWS, D), F32),
        pltpu.SemaphoreType.DMA,
    ],
)
def _sc_edge(xl, xr, idx3, dstc, dstdc, wv, av, *rest):
    _sc_edge_body(xl, xr, idx3, dstc, dstdc, wv, av, *rest)


def kernel(x, e, params, edge_index):
    idx3 = jnp.stack([
        edge_index[0].reshape(NW, NCHUNK, CHUNK),
        edge_index[1].reshape(NW, NCHUNK, CHUNK),
        lax.bitcast_convert_type(e[:, 0], jnp.int32).reshape(NW, NCHUNK, CHUNK),
    ], axis=2)
    dstc = edge_index[1].reshape(NW, NCHUNK, CHUNK)
    dstdc = (dstc >> 3).astype(jnp.int32)
    gat = params['gat']
    norm = params['norm']

    def r2(v):
        return v.reshape(1, D)

    h = x
    p = gat[0]
    xl, xr = _pre(h, p['Wl'], r2(p['bl']), p['Wr'], r2(p['br']))
    for i in range(4):
        p = gat[i]
        we = p['We'][0]
        num2, dpack = _sc_edge(xl, xr, idx3, dstc, dstdc, we, p['att'])
        den2 = dpack.reshape(N_PAD, 16)
        if i < 3:
            q = gat[i + 1]
            xl, xr = _combine(num2, den2, r2(p['bias']),
                              r2(norm[i]['gamma']), r2(norm[i]['beta']),
                              q['Wl'], r2(q['bl']), q['Wr'], r2(q['br']))
        else:
            return _final(num2, den2, r2(p['bias']))
